# Initial kernel scaffold; baseline (speedup 1.0000x reference)
import jax
import jax.numpy as jnp
from jax.experimental import pallas as pl


def kernel(pocket_x, ligand_x, pocket_edge_attr, ligand_edge_attr, complex_edge_attr, params, pocket_edge_index, ligand_edge_index, complex_edge_index, pocket_pos_idx, ligand_pos_idx, ligand_batch, atom_ymask_idx, pocket_ymask_idx):
    raise NotImplementedError("write your pallas kernel here")


# speedup vs baseline: 1.0829x; 1.0829x over previous
import functools

import jax
import jax.numpy as jnp
from jax import lax
from jax.experimental import pallas as pl
from jax.experimental.pallas import tpu as pltpu
from jax.experimental.pallas import tpu_sc as plsc

NC = 2
NS = 16
CHUNK = 128


def _leaky(x):
    return jnp.where(x > 0, x, 0.1 * x)


def _round_up(x, m):
    return (x + m - 1) // m * m


def _ea_body(attr_ref, we_ref, be_ref, out_ref):
    out_ref[...] = (
        jnp.dot(attr_ref[...], we_ref[...],
                preferred_element_type=jnp.float32)
        + be_ref[...])


@functools.lru_cache(maxsize=None)
def _make_ea(e_pad, de):
    be_blk = 2048
    grid = e_pad // be_blk
    return pl.pallas_call(
        _ea_body,
        grid=(grid,),
        in_specs=[
            pl.BlockSpec((be_blk, de), lambda i: (i, 0)),
            pl.BlockSpec((de, 128), lambda i: (0, 0)),
            pl.BlockSpec((1, 128), lambda i: (0, 0)),
        ],
        out_specs=pl.BlockSpec((be_blk, 128), lambda i: (i, 0)),
        out_shape=jax.ShapeDtypeStruct((e_pad, 128), jnp.float32),
    )


def _node_body(residual, x_ref, agg_ref, w1_ref, b1_ref, w2_ref, b2_ref,
               g_ref, bb_ref, out_ref):
    x = x_ref[...]
    t = x + agg_ref[...]
    u = _leaky(jnp.dot(t, w1_ref[...],
                       preferred_element_type=jnp.float32)
               + b1_ref[...])
    v = (jnp.dot(u, w2_ref[...],
                 preferred_element_type=jnp.float32)
         + b2_ref[...])
    mu = jnp.mean(v, axis=-1, keepdims=True)
    var = jnp.mean((v - mu) * (v - mu), axis=-1, keepdims=True)
    w = (v - mu) * lax.rsqrt(var + 1e-5) * g_ref[...] + bb_ref[...]
    if residual:
        w = w + x
    out_ref[...] = _leaky(w)


@functools.lru_cache(maxsize=None)
def _make_node(n, residual):
    bn = 1000
    grid = n // bn
    full = lambda i: (0, 0)
    return pl.pallas_call(
        functools.partial(_node_body, residual),
        grid=(grid,),
        in_specs=[
            pl.BlockSpec((bn, 128), lambda i: (i, 0)),
            pl.BlockSpec((bn, 128), lambda i: (i, 0)),
            pl.BlockSpec((128, 128), full),
            pl.BlockSpec((1, 128), full),
            pl.BlockSpec((128, 128), full),
            pl.BlockSpec((1, 128), full),
            pl.BlockSpec((1, 128), full),
            pl.BlockSpec((1, 128), full),
        ],
        out_specs=pl.BlockSpec((bn, 128), lambda i: (i, 0)),
        out_shape=jax.ShapeDtypeStruct((n, 128), jnp.float32),
    )


_SPMEM_BUDGET = 2_060_000


def _chunk_for(n):
    acc = (_round_up(n // 2, 128) + 128) * 128
    for chunk in (128, 64, 32):
        if acc + NS * (2 * chunk * 128 + 3 * chunk + 32) <= _SPMEM_BUDGET:
            return chunk
    raise ValueError(f"node count {n} too large for Spmem accumulator")


@functools.lru_cache(maxsize=None)
def _make_agg(n, e_pad, chunk):
    nh = n // 2
    nh_pad = _round_up(nh, 128)
    acc_rows = nh_pad + 128
    dump = nh_pad
    esh = e_pad // NS
    n_chunks = esh // chunk
    nzc = acc_rows // chunk
    noc = nh_pad // chunk
    mesh = plsc.VectorSubcoreMesh(core_axis_name="c", subcore_axis_name="s",
                                  num_cores=NC, num_subcores=NS)

    @functools.partial(
        pl.kernel,
        out_type=jax.ShapeDtypeStruct((2 * nh_pad, 128), jnp.float32),
        mesh=mesh,
        scratch_types=[
            pltpu.VMEM((chunk,), jnp.int32),
            pltpu.VMEM((chunk,), jnp.int32),
            pltpu.VMEM((chunk,), jnp.int32),
            pltpu.VMEM((chunk, 128), jnp.float32),
            pltpu.VMEM((chunk, 128), jnp.float32),
            pltpu.VMEM_SHARED((acc_rows, 128), jnp.float32),
            pltpu.SemaphoreType.DMA,
        ],
    )
    def agg_fn(x_hbm, ea_hbm, src_hbm, dst_hbm, perm_hbm, out_hbm,
               srcv, dstv, permv, xg, eav, acc, sem):
        c = lax.axis_index("c")
        s = lax.axis_index("s")
        base_node = c * nh

        def zrow(i, _):
            for j in range(8):
                xg[i, pl.ds(j * 16, 16)] = jnp.zeros((16,), jnp.float32)
            return 0
        lax.fori_loop(0, chunk, zrow, 0)

        def zchunk(j, _):
            k = j * NS + s

            @pl.when(k < nzc)
            def _():
                pltpu.sync_copy(xg, acc.at[pl.ds(k * chunk, chunk)])
            return 0
        lax.fori_loop(0, (nzc + NS - 1) // NS, zchunk, 0)
        plsc.subcore_barrier()

        def ebody(t, _):
            eb = s * esh + t * chunk
            pltpu.sync_copy(src_hbm.at[pl.ds(eb, chunk)], srcv)
            pltpu.sync_copy(dst_hbm.at[pl.ds(eb, chunk)], dstv)
            pltpu.sync_copy(perm_hbm.at[pl.ds(eb, chunk)], permv)
            pltpu.async_copy(x_hbm.at[srcv], xg, sem).wait()
            pltpu.async_copy(ea_hbm.at[permv], eav, sem).wait()

            for i in range(chunk // 16):
                sl = pl.ds(i * 16, 16)
                local = dstv[sl] - base_node
                ok = (local >= 0) & (local < nh)
                dstv[sl] = jnp.where(ok, local, dump)

            def mrow(i, _):
                for j in range(8):
                    sl = pl.ds(j * 16, 16)
                    xg[i, sl] = jnp.maximum(xg[i, sl] + eav[i, sl], 0.0)
                return 0
            lax.fori_loop(0, chunk, mrow, 0)

            pltpu.sync_copy(xg, acc.at[dstv], add=True)
            return 0
        lax.fori_loop(0, n_chunks, ebody, 0)
        plsc.subcore_barrier()

        def obody(j, _):
            k = j * NS + s

            @pl.when(k < noc)
            def _():
                pltpu.sync_copy(acc.at[pl.ds(k * chunk, chunk)], xg)
                pltpu.sync_copy(
                    xg, out_hbm.at[pl.ds(c * nh_pad + k * chunk, chunk)])
            return 0
        lax.fori_loop(0, (noc + NS - 1) // NS, obody, 0)

    return agg_fn, nh, nh_pad


def _head_body(lig_ref, batch_ref, wa1, ba1, wa2, ba2, wa3, ba3,
               wp1, bp1, wp2, bp2, wp3, bp3, atom_ref, pock_ref):
    lig = lig_ref[...]
    a = _leaky(jnp.dot(lig, wa1[...],
                       preferred_element_type=jnp.float32)
               + ba1[...])
    a = _leaky(jnp.dot(a, wa2[...],
                       preferred_element_type=jnp.float32)
               + ba2[...])
    atom_ref[...] = (jnp.dot(a, wa3[...],
                             preferred_element_type=jnp.float32) + ba3[...])

    nl = lig.shape[0]
    b = batch_ref[...]
    seg = lax.broadcasted_iota(jnp.int32, (64, nl), 0)
    mask = (seg == b).astype(jnp.float32)
    gsum = jnp.dot(mask, lig, preferred_element_type=jnp.float32,
                   precision=lax.Precision.HIGHEST)
    cnt = jnp.sum(mask, axis=1, keepdims=True)
    gmean = gsum / jnp.maximum(cnt, 1.0)
    p = _leaky(jnp.dot(gmean, wp1[...],
                       preferred_element_type=jnp.float32)
               + bp1[...])
    p = _leaky(jnp.dot(p, wp2[...],
                       preferred_element_type=jnp.float32)
               + bp2[...])
    pock_ref[...] = (jnp.dot(p, wp3[...],
                             preferred_element_type=jnp.float32) + bp3[...])


@functools.lru_cache(maxsize=None)
def _make_head(nl):
    return pl.pallas_call(
        _head_body,
        out_shape=(jax.ShapeDtypeStruct((nl, 1), jnp.float32),
                   jax.ShapeDtypeStruct((64, 1), jnp.float32)),
    )


def _row(v):
    return v.reshape(1, -1)


def _gnn_block(x, edge_index, edge_attr, layers, always_residual):
    n = x.shape[0]
    e = edge_attr.shape[0]
    chunk = _chunk_for(n)
    e_pad = _round_up(e, 2048)
    de = edge_attr.shape[1]
    pad_e = e_pad - e
    src = jnp.pad(edge_index[0], (0, pad_e))
    dst = jnp.pad(edge_index[1], (0, pad_e), constant_values=n)
    attr = jnp.pad(edge_attr, ((0, pad_e), (0, 0)))
    perm = jnp.argsort(dst, stable=True).astype(jnp.int32)
    src_s = src[perm]
    dst_s = dst[perm]
    ea_fn = _make_ea(e_pad, de)
    agg_fn, nh, nh_pad = _make_agg(n, e_pad, chunk)
    for d, lp in enumerate(layers):
        ea = ea_fn(attr, lp['We'], _row(lp['be']))
        agg2 = agg_fn(x, ea, src_s, dst_s, perm)
        agg = jnp.concatenate(
            [agg2[:nh], agg2[nh_pad:nh_pad + nh]], axis=0)
        node_fn = _make_node(n, bool(d > 0 or always_residual))
        x = node_fn(x, agg, lp['W1'], _row(lp['b1']), lp['W2'], _row(lp['b2']),
                    _row(lp['ln_g']), _row(lp['ln_b']))
    return x


def kernel(pocket_x, ligand_x, pocket_edge_attr, ligand_edge_attr,
           complex_edge_attr, params, pocket_edge_index, ligand_edge_index,
           complex_edge_index, pocket_pos_idx, ligand_pos_idx, ligand_batch,
           atom_ymask_idx, pocket_ymask_idx):
    np_ = pocket_x.shape[0]
    pf = _gnn_block(pocket_x, pocket_edge_index, pocket_edge_attr,
                    params['pocket'], False)
    lf = _gnn_block(ligand_x, ligand_edge_index, ligand_edge_attr,
                    params['ligand'], False)
    cf = jnp.concatenate([pf, lf], axis=0)
    cf = _gnn_block(cf, complex_edge_index, complex_edge_attr,
                    params['complex'], True)
    lig = cf[np_:]
    wa, wp = params['W_atom'], params['W_pocket']
    head_fn = _make_head(lig.shape[0])
    atom_full, pock_full = head_fn(
        lig, ligand_batch.reshape(1, -1),
        wa['W1'], _row(wa['b1']), wa['W2'], _row(wa['b2']),
        wa['W3'], _row(wa['b3']),
        wp['W1'], _row(wp['b1']), wp['W2'], _row(wp['b2']),
        wp['W3'], _row(wp['b3']))
    atom_pred = atom_full[atom_ymask_idx]
    pocket_pred = pock_full[pocket_ymask_idx]
    return pocket_pred, atom_pred


# speedup vs baseline: 1.4076x; 1.2998x over previous
import functools

import jax
import jax.numpy as jnp
from jax import lax
from jax.experimental import pallas as pl
from jax.experimental.pallas import tpu as pltpu
from jax.experimental.pallas import tpu_sc as plsc

NC = 2
NS = 16
CHUNK = 128


def _leaky(x):
    return jnp.where(x > 0, x, 0.1 * x)


def _round_up(x, m):
    return (x + m - 1) // m * m


def _ea_body(attr_ref, we_ref, be_ref, out_ref):
    out_ref[...] = (
        jnp.dot(attr_ref[...], we_ref[...],
                preferred_element_type=jnp.float32)
        + be_ref[...])


@functools.lru_cache(maxsize=None)
def _make_ea(e_pad, de):
    be_blk = 2048
    grid = e_pad // be_blk
    return pl.pallas_call(
        _ea_body,
        grid=(grid,),
        in_specs=[
            pl.BlockSpec((be_blk, de), lambda i: (i, 0)),
            pl.BlockSpec((de, 128), lambda i: (0, 0)),
            pl.BlockSpec((1, 128), lambda i: (0, 0)),
        ],
        out_specs=pl.BlockSpec((be_blk, 128), lambda i: (i, 0)),
        out_shape=jax.ShapeDtypeStruct((e_pad, 128), jnp.float32),
    )


def _node_body(residual, x_ref, agg_ref, w1_ref, b1_ref, w2_ref, b2_ref,
               g_ref, bb_ref, out_ref):
    x = x_ref[...]
    t = x + agg_ref[...]
    u = _leaky(jnp.dot(t, w1_ref[...],
                       preferred_element_type=jnp.float32)
               + b1_ref[...])
    v = (jnp.dot(u, w2_ref[...],
                 preferred_element_type=jnp.float32)
         + b2_ref[...])
    mu = jnp.mean(v, axis=-1, keepdims=True)
    var = jnp.mean((v - mu) * (v - mu), axis=-1, keepdims=True)
    w = (v - mu) * lax.rsqrt(var + 1e-5) * g_ref[...] + bb_ref[...]
    if residual:
        w = w + x
    out_ref[...] = _leaky(w)


@functools.lru_cache(maxsize=None)
def _make_node(n, residual):
    bn = 1000
    grid = n // bn
    full = lambda i: (0, 0)
    return pl.pallas_call(
        functools.partial(_node_body, residual),
        grid=(grid,),
        in_specs=[
            pl.BlockSpec((bn, 128), lambda i: (i, 0)),
            pl.BlockSpec((bn, 128), lambda i: (i, 0)),
            pl.BlockSpec((128, 128), full),
            pl.BlockSpec((1, 128), full),
            pl.BlockSpec((128, 128), full),
            pl.BlockSpec((1, 128), full),
            pl.BlockSpec((1, 128), full),
            pl.BlockSpec((1, 128), full),
        ],
        out_specs=pl.BlockSpec((bn, 128), lambda i: (i, 0)),
        out_shape=jax.ShapeDtypeStruct((n, 128), jnp.float32),
    )


_SPMEM_BUDGET = 2_060_000


def _chunk_for(n):
    acc = (_round_up(n // 2, 128) + 128) * 128
    for chunk in (128, 64, 32):
        if acc + NS * (2 * chunk * 128 + 3 * chunk + 32) <= _SPMEM_BUDGET:
            return chunk
    raise ValueError(f"node count {n} too large for Spmem accumulator")


@functools.lru_cache(maxsize=None)
def _make_agg(n, e_pad, chunk):
    nh = n // 2
    nh_pad = _round_up(nh, 128)
    acc_rows = nh_pad + 128
    dump = nh_pad
    esh = e_pad // NS
    n_chunks = esh // chunk
    nzc = acc_rows // chunk
    noc = nh_pad // chunk
    mesh = plsc.VectorSubcoreMesh(core_axis_name="c", subcore_axis_name="s",
                                  num_cores=NC, num_subcores=NS)

    @functools.partial(
        pl.kernel,
        out_type=jax.ShapeDtypeStruct((2 * nh_pad, 128), jnp.float32),
        mesh=mesh,
        scratch_types=[
            pltpu.VMEM((chunk,), jnp.int32),
            pltpu.VMEM((chunk,), jnp.int32),
            pltpu.VMEM((chunk,), jnp.int32),
            pltpu.VMEM((chunk, 128), jnp.float32),
            pltpu.VMEM((chunk, 128), jnp.float32),
            pltpu.VMEM_SHARED((acc_rows, 128), jnp.float32),
            pltpu.SemaphoreType.DMA,
            pltpu.SemaphoreType.DMA,
        ],
    )
    def agg_fn(x_hbm, ea_hbm, src_hbm, dst_hbm, perm_hbm, out_hbm,
               srcv, dstv, permv, xg, eav, acc, sem, sem2):
        c = lax.axis_index("c")
        s = lax.axis_index("s")
        base_node = c * nh

        def zrow(i, _):
            for j in range(8):
                xg[i, pl.ds(j * 16, 16)] = jnp.zeros((16,), jnp.float32)
            return 0
        lax.fori_loop(0, chunk, zrow, 0)

        def zchunk(j, _):
            k = j * NS + s

            @pl.when(k < nzc)
            def _():
                pltpu.sync_copy(xg, acc.at[pl.ds(k * chunk, chunk)])
            return 0
        lax.fori_loop(0, (nzc + NS - 1) // NS, zchunk, 0)
        plsc.subcore_barrier()

        def ebody(t, _):
            eb = s * esh + t * chunk
            d1 = pltpu.async_copy(src_hbm.at[pl.ds(eb, chunk)], srcv, sem)
            d2 = pltpu.async_copy(dst_hbm.at[pl.ds(eb, chunk)], dstv, sem)
            d3 = pltpu.async_copy(perm_hbm.at[pl.ds(eb, chunk)], permv, sem)
            d1.wait()
            d2.wait()
            d3.wait()
            g1 = pltpu.async_copy(x_hbm.at[srcv], xg, sem)
            g2 = pltpu.async_copy(ea_hbm.at[permv], eav, sem2)
            g1.wait()
            g2.wait()

            for i in range(chunk // 16):
                sl = pl.ds(i * 16, 16)
                local = dstv[sl] - base_node
                ok = (local >= 0) & (local < nh)
                dstv[sl] = jnp.where(ok, local, dump)

            def mrow(i, _):
                for j in range(8):
                    sl = pl.ds(j * 16, 16)
                    xg[i, sl] = jnp.maximum(xg[i, sl] + eav[i, sl], 0.0)
                return 0
            lax.fori_loop(0, chunk, mrow, 0)

            pltpu.sync_copy(xg, acc.at[dstv], add=True)
            return 0
        lax.fori_loop(0, n_chunks, ebody, 0)
        plsc.subcore_barrier()

        def obody(j, _):
            k = j * NS + s

            @pl.when(k < noc)
            def _():
                pltpu.sync_copy(acc.at[pl.ds(k * chunk, chunk)], xg)
                pltpu.sync_copy(
                    xg, out_hbm.at[pl.ds(c * nh_pad + k * chunk, chunk)])
            return 0
        lax.fori_loop(0, (noc + NS - 1) // NS, obody, 0)

    return agg_fn, nh, nh_pad


def _head_body(lig_ref, batch_ref, wa1, ba1, wa2, ba2, wa3, ba3,
               wp1, bp1, wp2, bp2, wp3, bp3, atom_ref, pock_ref):
    lig = lig_ref[...]
    a = _leaky(jnp.dot(lig, wa1[...],
                       preferred_element_type=jnp.float32)
               + ba1[...])
    a = _leaky(jnp.dot(a, wa2[...],
                       preferred_element_type=jnp.float32)
               + ba2[...])
    atom_ref[...] = (jnp.dot(a, wa3[...],
                             preferred_element_type=jnp.float32) + ba3[...])

    nl = lig.shape[0]
    b = batch_ref[...]
    seg = lax.broadcasted_iota(jnp.int32, (64, nl), 0)
    mask = (seg == b).astype(jnp.float32)
    gsum = jnp.dot(mask, lig, preferred_element_type=jnp.float32,
                   precision=lax.Precision.HIGHEST)
    cnt = jnp.sum(mask, axis=1, keepdims=True)
    gmean = gsum / jnp.maximum(cnt, 1.0)
    p = _leaky(jnp.dot(gmean, wp1[...],
                       preferred_element_type=jnp.float32)
               + bp1[...])
    p = _leaky(jnp.dot(p, wp2[...],
                       preferred_element_type=jnp.float32)
               + bp2[...])
    pock_ref[...] = (jnp.dot(p, wp3[...],
                             preferred_element_type=jnp.float32) + bp3[...])


@functools.lru_cache(maxsize=None)
def _make_head(nl):
    return pl.pallas_call(
        _head_body,
        out_shape=(jax.ShapeDtypeStruct((nl, 1), jnp.float32),
                   jax.ShapeDtypeStruct((64, 1), jnp.float32)),
    )


def _row(v):
    return v.reshape(1, -1)


def _gnn_block(x, edge_index, edge_attr, layers, always_residual):
    n = x.shape[0]
    e = edge_attr.shape[0]
    chunk = _chunk_for(n)
    e_pad = _round_up(e, 2048)
    de = edge_attr.shape[1]
    pad_e = e_pad - e
    src = jnp.pad(edge_index[0], (0, pad_e))
    dst = jnp.pad(edge_index[1], (0, pad_e), constant_values=n)
    attr = jnp.pad(edge_attr, ((0, pad_e), (0, 0)))
    perm = jnp.argsort(dst, stable=True).astype(jnp.int32)
    src_s = src[perm]
    dst_s = dst[perm]
    ea_fn = _make_ea(e_pad, de)
    agg_fn, nh, nh_pad = _make_agg(n, e_pad, chunk)
    for d, lp in enumerate(layers):
        ea = ea_fn(attr, lp['We'], _row(lp['be']))
        agg2 = agg_fn(x, ea, src_s, dst_s, perm)
        agg = jnp.concatenate(
            [agg2[:nh], agg2[nh_pad:nh_pad + nh]], axis=0)
        node_fn = _make_node(n, bool(d > 0 or always_residual))
        x = node_fn(x, agg, lp['W1'], _row(lp['b1']), lp['W2'], _row(lp['b2']),
                    _row(lp['ln_g']), _row(lp['ln_b']))
    return x


def kernel(pocket_x, ligand_x, pocket_edge_attr, ligand_edge_attr,
           complex_edge_attr, params, pocket_edge_index, ligand_edge_index,
           complex_edge_index, pocket_pos_idx, ligand_pos_idx, ligand_batch,
           atom_ymask_idx, pocket_ymask_idx):
    np_ = pocket_x.shape[0]
    pf = _gnn_block(pocket_x, pocket_edge_index, pocket_edge_attr,
                    params['pocket'], False)
    lf = _gnn_block(ligand_x, ligand_edge_index, ligand_edge_attr,
                    params['ligand'], False)
    cf = jnp.concatenate([pf, lf], axis=0)
    cf = _gnn_block(cf, complex_edge_index, complex_edge_attr,
                    params['complex'], True)
    lig = cf[np_:]
    wa, wp = params['W_atom'], params['W_pocket']
    head_fn = _make_head(lig.shape[0])
    atom_full, pock_full = head_fn(
        lig, ligand_batch.reshape(1, -1),
        wa['W1'], _row(wa['b1']), wa['W2'], _row(wa['b2']),
        wa['W3'], _row(wa['b3']),
        wp['W1'], _row(wp['b1']), wp['W2'], _row(wp['b2']),
        wp['W3'], _row(wp['b3']))
    atom_pred = atom_full[atom_ymask_idx]
    pocket_pred = pock_full[pocket_ymask_idx]
    return pocket_pred, atom_pred
